# Initial kernel scaffold; baseline (speedup 1.0000x reference)
import jax
import jax.numpy as jnp
from jax.experimental import pallas as pl


def kernel(xyz1, xyz2, points1, points2, W0, W1):
    raise NotImplementedError("write your pallas kernel here")


# speedup vs baseline: 41.7115x; 41.7115x over previous
import functools

import jax
import jax.numpy as jnp
from jax.experimental import pallas as pl


def _p2w_body(p2_ref, w0a_ref, out_ref):
    out_ref[0] = jnp.dot(p2_ref[0], w0a_ref[...],
                         preferred_element_type=jnp.float32)


def _main_body(x1_ref, x2_ref, p1_ref, p2w_ref, w0b_ref, w1_ref, out_ref):
    x1 = x1_ref[0]
    x2 = x2_ref[0]
    d2 = ((x1[:, 0:1] - x2[0:1, :]) ** 2
          + (x1[:, 1:2] - x2[1:2, :]) ** 2
          + (x1[:, 2:3] - x2[2:3, :]) ** 2)
    big = jnp.float32(3.0e38)
    m1 = jnp.min(d2, axis=1, keepdims=True)
    d2a = jnp.where(d2 <= m1, big, d2)
    m2 = jnp.min(d2a, axis=1, keepdims=True)
    d2b = jnp.where(d2a <= m2, big, d2a)
    m3 = jnp.min(d2b, axis=1, keepdims=True)
    mask = d2 <= m3
    w = jnp.where(mask, 1.0 / jnp.maximum(d2, 1e-10), 0.0)
    wn = w / jnp.sum(w, axis=1, keepdims=True)
    hpart = jnp.dot(wn, p2w_ref[0], preferred_element_type=jnp.float32)
    h = jnp.maximum(
        hpart + jnp.dot(p1_ref[0], w0b_ref[...],
                        preferred_element_type=jnp.float32), 0.0)
    out_ref[0] = jnp.maximum(
        jnp.dot(h, w1_ref[...], preferred_element_type=jnp.float32), 0.0)


@jax.jit
def kernel(xyz1, xyz2, points1, points2, W0, W1):
    B, N, _ = xyz1.shape
    M = xyz2.shape[1]
    C1 = points1.shape[2]
    C2 = points2.shape[2]
    F0 = W0.shape[1]
    F1 = W1.shape[1]
    W0a = W0[:C2]
    W0b = W0[C2:]
    xyz2t = jnp.transpose(xyz2, (0, 2, 1))

    p2w = pl.pallas_call(
        _p2w_body,
        grid=(B,),
        in_specs=[
            pl.BlockSpec((1, M, C2), lambda b: (b, 0, 0)),
            pl.BlockSpec((C2, F0), lambda b: (0, 0)),
        ],
        out_specs=pl.BlockSpec((1, M, F0), lambda b: (b, 0, 0)),
        out_shape=jax.ShapeDtypeStruct((B, M, F0), jnp.float32),
    )(points2, W0a)

    BN = 512
    out = pl.pallas_call(
        _main_body,
        grid=(B, N // BN),
        in_specs=[
            pl.BlockSpec((1, BN, 3), lambda b, n: (b, n, 0)),
            pl.BlockSpec((1, 3, M), lambda b, n: (b, 0, 0)),
            pl.BlockSpec((1, BN, C1), lambda b, n: (b, n, 0)),
            pl.BlockSpec((1, M, F0), lambda b, n: (b, 0, 0)),
            pl.BlockSpec((C1, F0), lambda b, n: (0, 0)),
            pl.BlockSpec((F0, F1), lambda b, n: (0, 0)),
        ],
        out_specs=pl.BlockSpec((1, BN, F1), lambda b, n: (b, n, 0)),
        out_shape=jax.ShapeDtypeStruct((B, N, F1), jnp.float32),
    )(xyz1, xyz2t, points1, p2w, W0b, W1)
    return out
